# baseline (speedup 1.0000x reference)
import functools

import jax
import jax.numpy as jnp
from jax import lax
from jax.experimental import pallas as pl
from jax.experimental.pallas import tpu as pltpu
from jax.experimental.pallas import tpu_sc as plsc

B = 1024
E = 32
V = 100000
VT = 4096


@functools.cache
def _sc_gather_fn():
    info = plsc.get_sparse_core_info()
    nw = info.num_cores * info.num_subcores
    b_per_w = B // nw
    mesh = plsc.VectorSubcoreMesh(core_axis_name="c", subcore_axis_name="s")

    @functools.partial(
        pl.kernel,
        mesh=mesh,
        out_type=jax.ShapeDtypeStruct((B, E), jnp.float32),
        scratch_types=[
            pltpu.VMEM((b_per_w,), jnp.int32),
            pltpu.VMEM((b_per_w, E), jnp.float32),
            pltpu.SemaphoreType.DMA,
        ],
        compiler_params=pltpu.CompilerParams(use_tc_tiling_on_sc=False),
    )
    def gather(table_hbm, idx_hbm, out_hbm, idx_v, rows_v, sem):
        wid = lax.axis_index("s") * info.num_cores + lax.axis_index("c")
        base = wid * b_per_w
        pltpu.sync_copy(idx_hbm.at[pl.ds(base, b_per_w)], idx_v)
        pltpu.async_copy(table_hbm.at[idx_v], rows_v, sem).wait()
        pltpu.sync_copy(rows_v, out_hbm.at[pl.ds(base, b_per_w)])

    return gather


def _decoder_body(wt_ref, emb_ref, b_ref, out_ref):
    acc = jax.lax.dot_general(
        wt_ref[...], emb_ref[...], (((0,), (1,)), ((), ())),
        preferred_element_type=jnp.float32,
    )
    out_ref[...] = acc + jnp.reshape(b_ref[...], (VT, 1))


def _decoder_t(emb, Wt, b2):
    return pl.pallas_call(
        _decoder_body,
        grid=(pl.cdiv(V, VT),),
        in_specs=[
            pl.BlockSpec((E, VT), lambda i: (0, i)),
            pl.BlockSpec((B, E), lambda i: (0, 0)),
            pl.BlockSpec((1, VT), lambda i: (0, i)),
        ],
        out_specs=pl.BlockSpec((VT, B), lambda i: (i, 0)),
        out_shape=jax.ShapeDtypeStruct((V, B), jnp.float32),
    )(Wt, emb, b2)


def kernel(input_ids, emb_table, W, b):
    ids = input_ids.astype(jnp.int32)
    emb = _sc_gather_fn()(emb_table, ids)
    return _decoder_t(emb, W.T, b.reshape(1, V)).T


# speedup vs baseline: 1.0032x; 1.0032x over previous
import functools

import jax
import jax.numpy as jnp
from jax import lax
from jax.experimental import pallas as pl
from jax.experimental.pallas import tpu as pltpu
from jax.experimental.pallas import tpu_sc as plsc

B = 1024
E = 32
V = 100000
G = V // 4
VT = 4096


@functools.cache
def _sc_gather_fn():
    info = plsc.get_sparse_core_info()
    nw = info.num_cores * info.num_subcores
    b_per_w = B // nw
    mesh = plsc.VectorSubcoreMesh(core_axis_name="c", subcore_axis_name="s")

    @functools.partial(
        pl.kernel,
        mesh=mesh,
        out_type=jax.ShapeDtypeStruct((B, 128), jnp.float32),
        scratch_types=[
            pltpu.VMEM((b_per_w,), jnp.int32),
            pltpu.VMEM((b_per_w, 128), jnp.float32),
            pltpu.SemaphoreType.DMA,
        ],
    )
    def gather(table4_hbm, idx_hbm, out_hbm, idx_v, rows_v, sem):
        wid = lax.axis_index("s") * info.num_cores + lax.axis_index("c")
        base = wid * b_per_w
        pltpu.sync_copy(idx_hbm.at[pl.ds(base, b_per_w)], idx_v)
        pltpu.async_copy(table4_hbm.at[idx_v], rows_v, sem).wait()
        pltpu.sync_copy(rows_v, out_hbm.at[pl.ds(base, b_per_w)])

    return gather


def _decoder_body(wt_ref, emb4_ref, rem_ref, b_ref, out_ref, emb_s):
    @pl.when(pl.program_id(0) == 0)
    def _():
        rem = rem_ref[...]
        acc = jnp.zeros((B, E), jnp.float32)
        for r in range(4):
            acc += jnp.where(rem == r, 1.0, 0.0) * emb4_ref[:, r * E:(r + 1) * E]
        emb_s[...] = acc

    acc = jax.lax.dot_general(
        wt_ref[...], emb_s[...], (((0,), (1,)), ((), ())),
        preferred_element_type=jnp.float32,
    )
    out_ref[...] = acc + jnp.reshape(b_ref[...], (VT, 1))


def _decoder_t(emb4, rem2, Wt, b2):
    return pl.pallas_call(
        _decoder_body,
        grid=(pl.cdiv(V, VT),),
        in_specs=[
            pl.BlockSpec((E, VT), lambda i: (0, i)),
            pl.BlockSpec((B, 128), lambda i: (0, 0)),
            pl.BlockSpec((B, 1), lambda i: (0, 0)),
            pl.BlockSpec((1, VT), lambda i: (0, i)),
        ],
        out_specs=pl.BlockSpec((VT, B), lambda i: (i, 0)),
        out_shape=jax.ShapeDtypeStruct((V, B), jnp.float32),
        scratch_shapes=[pltpu.VMEM((B, E), jnp.float32)],
    )(Wt, emb4, rem2, b2)


def kernel(input_ids, emb_table, W, b):
    ids = input_ids.astype(jnp.int32)
    table4 = emb_table.reshape(G, 128)
    emb4 = _sc_gather_fn()(table4, ids >> 2)
    rem2 = (ids & 3).reshape(B, 1)
    return _decoder_t(emb4, rem2, W.T, b.reshape(1, V)).T


# speedup vs baseline: 1.1883x; 1.1846x over previous
import functools

import jax
import jax.numpy as jnp
from jax import lax
from jax.experimental import pallas as pl
from jax.experimental.pallas import tpu as pltpu
from jax.experimental.pallas import tpu_sc as plsc

B = 1024
E = 32
V = 100000
VT = 4096
_CHUNK = 128


@functools.cache
def _sc_gather_fn():
    info = plsc.get_sparse_core_info()
    nw = info.num_cores * info.num_subcores
    mesh = plsc.VectorSubcoreMesh(core_axis_name="c", subcore_axis_name="s")

    @functools.partial(
        pl.kernel,
        mesh=mesh,
        out_type=jax.ShapeDtypeStruct((E, B), jnp.float32),
        scratch_types=[
            pltpu.VMEM((B,), jnp.int32),
            pltpu.VMEM((B,), jnp.float32),
            pltpu.SemaphoreType.DMA,
        ],
        compiler_params=pltpu.CompilerParams(use_tc_tiling_on_sc=False),
    )
    def gather(tablet_hbm, idx_hbm, out_hbm, idx_v, row_v, sem):
        e = lax.axis_index("s") * info.num_cores + lax.axis_index("c")
        pltpu.sync_copy(idx_hbm, idx_v)
        copies = []
        for c in range(B // _CHUNK):
            copies.append(pltpu.async_copy(
                tablet_hbm.at[e].at[idx_v.at[pl.ds(c * _CHUNK, _CHUNK)]],
                row_v.at[pl.ds(c * _CHUNK, _CHUNK)],
                sem,
            ))
        for cp in copies:
            cp.wait()
        pltpu.sync_copy(row_v, out_hbm.at[e])

    return gather


def _decoder_body(wt_ref, embt_ref, b_ref, out_ref):
    acc = jax.lax.dot_general(
        wt_ref[...], embt_ref[...], (((0,), (0,)), ((), ())),
        preferred_element_type=jnp.float32,
    )
    out_ref[...] = acc + jnp.reshape(b_ref[...], (VT, 1))


def _decoder_t(embt, Wt, b2):
    return pl.pallas_call(
        _decoder_body,
        grid=(pl.cdiv(V, VT),),
        in_specs=[
            pl.BlockSpec((E, VT), lambda i: (0, i)),
            pl.BlockSpec((E, B), lambda i: (0, 0)),
            pl.BlockSpec((1, VT), lambda i: (0, i)),
        ],
        out_specs=pl.BlockSpec((VT, B), lambda i: (i, 0)),
        out_shape=jax.ShapeDtypeStruct((V, B), jnp.float32),
    )(Wt, embt, b2)


def kernel(input_ids, emb_table, W, b):
    ids = input_ids.astype(jnp.int32)
    embt = _sc_gather_fn()(emb_table.T, ids)
    return _decoder_t(embt, W.T, b.reshape(1, V)).T
